# baseline (speedup 1.0000x reference)
import functools

import jax
import jax.numpy as jnp
from jax import lax
from jax.experimental import pallas as pl
from jax.experimental.pallas import tpu as pltpu
from jax.experimental.pallas import tpu_sc as plsc

D = 32
NC = 2
NS = 16
NW = NC * NS
UNIT = 128
NBUF = 4


@functools.partial(jax.jit, static_argnames=("n_hist", "n_batch"))
def _sc_embed(table, idx_hm, *, n_hist, n_batch):
    nb0 = n_batch // UNIT
    n_units = n_hist * nb0
    upw = n_units // NW
    slab = D * n_batch
    out_len = n_hist * slab
    n_blocks = upw // NBUF
    mesh = plsc.VectorSubcoreMesh(core_axis_name="c", subcore_axis_name="s")

    @functools.partial(
        pl.kernel,
        out_type=jax.ShapeDtypeStruct((out_len,), jnp.float32),
        mesh=mesh,
        scratch_types=(
            [pltpu.VMEM((upw * UNIT,), jnp.int32),
             pltpu.VMEM((NBUF, UNIT, D), jnp.float32),
             pltpu.VMEM((NBUF, UNIT * D), jnp.float32)]
            + [pltpu.SemaphoreType.DMA] * (2 * NBUF)
        ),
        compiler_params=pltpu.CompilerParams(
            use_tc_tiling_on_sc=False, needs_layout_passes=False
        ),
    )
    def k(table_hbm, idx_hbm, out_hbm, idx_v, rows_v, tbuf, *sems):
        gs = sems[:NBUF]
        ws = sems[NBUF:]
        wid = lax.axis_index("s") * NC + lax.axis_index("c")
        ubase = wid * upw
        pltpu.sync_copy(idx_hbm.at[pl.ds(ubase * UNIT, upw * UNIT)], idx_v)

        iota = lax.iota(jnp.int32, 16)

        def gather(lu, s):
            return pltpu.make_async_copy(
                table_hbm.at[idx_v.at[pl.ds(lu * UNIT, UNIT)]],
                rows_v.at[s],
                gs[s],
            )

        def wcopy(lu, s, e0):
            u = ubase + lu
            h = u // nb0
            b0 = u % nb0
            off = h * slab + (e0 * nb0 + b0) * 1024
            return pltpu.make_async_copy(
                tbuf.at[s, pl.ds(e0 * 1024, 1024)],
                out_hbm.at[pl.ds(pl.multiple_of(off, 1024), 1024)],
                ws[s],
            )

        def block(g, carry):
            for s in range(NBUF):
                lu = g * NBUF + s

                @pl.when(g > 0)
                def _():
                    for e0 in range(D // 8):
                        wcopy(lu - NBUF, s, e0).wait()

                gather(lu, s).start()
            for s in range(NBUF):
                lu = g * NBUF + s
                gather(lu, s).wait()
                rows = rows_v.at[s]
                for e in range(D):
                    cols = jnp.full((16,), e, jnp.int32)
                    for j in range(UNIT // 16):
                        vals = plsc.load_gather(rows, [iota + j * 16, cols])
                        tbuf[s, pl.ds(e * UNIT + j * 16, 16)] = vals
                for e0 in range(D // 8):
                    wcopy(lu, s, e0).start()
            return carry

        lax.fori_loop(0, n_blocks, block, 0)
        for s in range(NBUF):
            for e0 in range(D // 8):
                wcopy((n_blocks - 1) * NBUF + s, s, e0).wait()

    return k(table, idx_hm)


def kernel(inputs, table):
    batch, hist = inputs.shape
    idx_hm = inputs.T.reshape(batch * hist).astype(jnp.int32)
    out_flat = _sc_embed(table, idx_hm, n_hist=hist, n_batch=batch)
    out5 = out_flat.reshape(hist, D // 8, batch // UNIT, 8, UNIT)
    return out5.transpose(2, 4, 0, 1, 3).reshape(batch, hist, D)


# speedup vs baseline: 1.3699x; 1.3699x over previous
import functools

import jax
import jax.numpy as jnp
from jax import lax
from jax.experimental import pallas as pl
from jax.experimental.pallas import tpu as pltpu
from jax.experimental.pallas import tpu_sc as plsc

D = 32
NC = 2
NS = 16
NW = NC * NS
UNIT = 128
NBUF = 4


@functools.partial(jax.jit, static_argnames=("n_hist", "n_batch"))
def _sc_embed(table, idx_hm, *, n_hist, n_batch):
    nb0 = n_batch // UNIT
    n_units = n_hist * nb0
    upw = n_units // NW
    slab = D * n_batch
    out_len = n_hist * slab
    n_blocks = upw // NBUF
    mesh = plsc.VectorSubcoreMesh(core_axis_name="c", subcore_axis_name="s")

    @functools.partial(
        pl.kernel,
        out_type=jax.ShapeDtypeStruct((out_len,), jnp.float32),
        mesh=mesh,
        scratch_types=(
            [pltpu.VMEM((upw * UNIT,), jnp.int32),
             pltpu.VMEM((NBUF, UNIT, D), jnp.float32),
             pltpu.VMEM((NBUF, UNIT * D), jnp.float32)]
            + [pltpu.SemaphoreType.DMA] * (2 * NBUF)
        ),
        compiler_params=pltpu.CompilerParams(
            use_tc_tiling_on_sc=False, needs_layout_passes=False
        ),
    )
    def k(table_hbm, idx_hbm, out_hbm, idx_v, rows_v, tbuf, *sems):
        gs = sems[:NBUF]
        ws = sems[NBUF:]
        wid = lax.axis_index("s") * NC + lax.axis_index("c")
        ubase = wid * upw
        pltpu.sync_copy(idx_hbm.at[pl.ds(ubase * UNIT, upw * UNIT)], idx_v)

        iota = lax.iota(jnp.int32, 16)
        rowidx = [iota + j * 16 for j in range(UNIT // 16)]
        colidx = [jnp.full((16,), e, jnp.int32) for e in range(D)]

        def gather(lu, s):
            return pltpu.make_async_copy(
                table_hbm.at[idx_v.at[pl.ds(lu * UNIT, UNIT)]],
                rows_v.at[s],
                gs[s],
            )

        def wcopy(lu, s, e0):
            u = ubase + lu
            h = u // nb0
            b0 = u % nb0
            off = h * slab + (e0 * nb0 + b0) * 1024
            return pltpu.make_async_copy(
                tbuf.at[s, pl.ds(e0 * 1024, 1024)],
                out_hbm.at[pl.ds(pl.multiple_of(off, 1024), 1024)],
                ws[s],
            )

        def block(g, carry):
            for s in range(NBUF):
                lu = g * NBUF + s

                @pl.when(g > 0)
                def _():
                    for e0 in range(D // 8):
                        wcopy(lu - NBUF, s, e0).wait()

                gather(lu, s).start()
            for s in range(NBUF):
                lu = g * NBUF + s
                gather(lu, s).wait()
                rows = rows_v.at[s]
                for j in range(UNIT // 16):
                    vals = [
                        plsc.load_gather(rows, [rowidx[j], colidx[e]])
                        for e in range(D)
                    ]
                    for e in range(D):
                        tbuf[s, pl.ds(e * UNIT + j * 16, 16)] = vals[e]
                for e0 in range(D // 8):
                    wcopy(lu, s, e0).start()
            return carry

        lax.fori_loop(0, n_blocks, block, 0)
        for s in range(NBUF):
            for e0 in range(D // 8):
                wcopy((n_blocks - 1) * NBUF + s, s, e0).wait()

    return k(table, idx_hm)


def kernel(inputs, table):
    batch, hist = inputs.shape
    idx_hm = inputs.T.reshape(batch * hist).astype(jnp.int32)
    out_flat = _sc_embed(table, idx_hm, n_hist=hist, n_batch=batch)
    out5 = out_flat.reshape(hist, D // 8, batch // UNIT, 8, UNIT)
    return out5.transpose(2, 4, 0, 1, 3).reshape(batch, hist, D)


# speedup vs baseline: 2.0147x; 1.4708x over previous
import functools

import jax
import jax.numpy as jnp
from jax import lax
from jax.experimental import pallas as pl
from jax.experimental.pallas import tpu as pltpu
from jax.experimental.pallas import tpu_sc as plsc

D = 32
NC = 2
NS = 16
NW = NC * NS
UNIT = 128
NBUF = 4


@functools.partial(jax.jit, static_argnames=("n_hist", "n_batch"))
def _sc_embed(table, idx_hm, *, n_hist, n_batch):
    nb0 = n_batch // UNIT
    n_units = n_hist * nb0
    upw = n_units // NW
    slab = D * n_batch
    out_len = n_hist * slab
    n_blocks = upw // NBUF
    mesh = plsc.VectorSubcoreMesh(core_axis_name="c", subcore_axis_name="s")

    @functools.partial(
        pl.kernel,
        out_type=jax.ShapeDtypeStruct((out_len,), jnp.float32),
        mesh=mesh,
        scratch_types=(
            [pltpu.VMEM((upw * UNIT,), jnp.int32),
             pltpu.VMEM((NBUF, UNIT, D), jnp.float32),
             pltpu.VMEM((NBUF, UNIT * D), jnp.float32)]
            + [pltpu.SemaphoreType.DMA] * (2 * NBUF)
        ),
        compiler_params=pltpu.CompilerParams(
            use_tc_tiling_on_sc=False, needs_layout_passes=False
        ),
    )
    def k(table_hbm, idx_hbm, out_hbm, idx_v, rows_v, tbuf, *sems):
        gs = sems[:NBUF]
        ws = sems[NBUF:]
        wid = lax.axis_index("s") * NC + lax.axis_index("c")
        ubase = wid * upw
        pltpu.sync_copy(idx_hbm.at[pl.ds(ubase * UNIT, upw * UNIT)], idx_v)

        iota = lax.iota(jnp.int32, 16)
        rowidx = [iota + j * 16 for j in range(UNIT // 16)]

        def gather(lu, s):
            return pltpu.make_async_copy(
                table_hbm.at[idx_v.at[pl.ds(lu * UNIT, UNIT)]],
                rows_v.at[s],
                gs[s],
            )

        def wcopy(lu, s, e0):
            u = ubase + lu
            h = u // nb0
            b0 = u % nb0
            off = h * slab + (e0 * nb0 + b0) * 1024
            return pltpu.make_async_copy(
                tbuf.at[s, pl.ds(e0 * 1024, 1024)],
                out_hbm.at[pl.ds(pl.multiple_of(off, 1024), 1024)],
                ws[s],
            )

        def block(g, carry):
            for s in range(NBUF):
                lu = g * NBUF + s

                @pl.when(g > 0)
                def _():
                    for e0 in range(D // 8):
                        wcopy(lu - NBUF, s, e0).wait()

                gather(lu, s).start()
            for s in range(NBUF):
                lu = g * NBUF + s
                gather(lu, s).wait()
                rows = rows_v.at[s]
                tbufs = tbuf.at[s]

                def tr_body(eb, carry):
                    for ee in range(8):
                        cole = (iota + eb * 8 + ee) & (D - 1)
                        scol = cole * UNIT + iota
                        loads = [
                            plsc.load_gather(rows, [rowidx[j], cole])
                            for j in range(UNIT // 16)
                        ]
                        for j in range(UNIT // 16):
                            plsc.store_scatter(
                                tbufs, [scol + j * 16], loads[j]
                            )
                    return carry

                lax.fori_loop(0, D // 8, tr_body, 0)
                for e0 in range(D // 8):
                    wcopy(lu, s, e0).start()
            return carry

        lax.fori_loop(0, n_blocks, block, 0)
        for s in range(NBUF):
            for e0 in range(D // 8):
                wcopy((n_blocks - 1) * NBUF + s, s, e0).wait()

    return k(table, idx_hm)


def kernel(inputs, table):
    batch, hist = inputs.shape
    idx_hm = inputs.T.reshape(batch * hist).astype(jnp.int32)
    out_flat = _sc_embed(table, idx_hm, n_hist=hist, n_batch=batch)
    out5 = out_flat.reshape(hist, D // 8, batch // UNIT, 8, UNIT)
    return out5.transpose(2, 4, 0, 1, 3).reshape(batch, hist, D)


# speedup vs baseline: 2.0686x; 1.0267x over previous
import functools

import jax
import jax.numpy as jnp
from jax import lax
from jax.experimental import pallas as pl
from jax.experimental.pallas import tpu as pltpu
from jax.experimental.pallas import tpu_sc as plsc

D = 32
NC = 2
NS = 16
NW = NC * NS
UNIT = 128
NBUF = 8


@functools.partial(jax.jit, static_argnames=("n_hist", "n_batch"))
def _sc_embed(table, idx_hm, *, n_hist, n_batch):
    nb0 = n_batch // UNIT
    n_units = n_hist * nb0
    upw = n_units // NW
    slab = D * n_batch
    out_len = n_hist * slab
    n_blocks = upw // NBUF
    mesh = plsc.VectorSubcoreMesh(core_axis_name="c", subcore_axis_name="s")

    @functools.partial(
        pl.kernel,
        out_type=jax.ShapeDtypeStruct((out_len,), jnp.float32),
        mesh=mesh,
        scratch_types=(
            [pltpu.VMEM((upw * UNIT,), jnp.int32),
             pltpu.VMEM((NBUF, UNIT, D), jnp.float32),
             pltpu.VMEM((NBUF, UNIT * D), jnp.float32)]
            + [pltpu.SemaphoreType.DMA] * (2 * NBUF)
        ),
        compiler_params=pltpu.CompilerParams(
            use_tc_tiling_on_sc=False, needs_layout_passes=False
        ),
    )
    def k(table_hbm, idx_hbm, out_hbm, idx_v, rows_v, tbuf, *sems):
        gs = sems[:NBUF]
        ws = sems[NBUF:]
        wid = lax.axis_index("s") * NC + lax.axis_index("c")
        ubase = wid * upw
        pltpu.sync_copy(idx_hbm.at[pl.ds(ubase * UNIT, upw * UNIT)], idx_v)

        iota = lax.iota(jnp.int32, 16)
        rowidx = [iota + j * 16 for j in range(UNIT // 16)]

        def gather(lu, s):
            return pltpu.make_async_copy(
                table_hbm.at[idx_v.at[pl.ds(lu * UNIT, UNIT)]],
                rows_v.at[s],
                gs[s],
            )

        def wcopy(lu, s, e0):
            u = ubase + lu
            h = u // nb0
            b0 = u % nb0
            off = h * slab + (e0 * nb0 + b0) * 1024
            return pltpu.make_async_copy(
                tbuf.at[s, pl.ds(e0 * 1024, 1024)],
                out_hbm.at[pl.ds(pl.multiple_of(off, 1024), 1024)],
                ws[s],
            )

        def block(g, carry):
            for s in range(NBUF):
                lu = g * NBUF + s

                @pl.when(g > 0)
                def _():
                    for e0 in range(D // 8):
                        wcopy(lu - NBUF, s, e0).wait()

                gather(lu, s).start()
            for s in range(NBUF):
                lu = g * NBUF + s
                gather(lu, s).wait()
                rows = rows_v.at[s]
                tbufs = tbuf.at[s]

                def tr_body(eb, carry):
                    for ee in range(8):
                        cole = (iota + eb * 8 + ee) & (D - 1)
                        scol = cole * UNIT + iota
                        loads = [
                            plsc.load_gather(rows, [rowidx[j], cole])
                            for j in range(UNIT // 16)
                        ]
                        for j in range(UNIT // 16):
                            plsc.store_scatter(
                                tbufs, [scol + j * 16], loads[j]
                            )
                    return carry

                lax.fori_loop(0, D // 8, tr_body, 0)
                for e0 in range(D // 8):
                    wcopy(lu, s, e0).start()
            return carry

        lax.fori_loop(0, n_blocks, block, 0)
        for s in range(NBUF):
            for e0 in range(D // 8):
                wcopy((n_blocks - 1) * NBUF + s, s, e0).wait()

    return k(table, idx_hm)


def kernel(inputs, table):
    batch, hist = inputs.shape
    idx_hm = inputs.T.reshape(batch * hist).astype(jnp.int32)
    out_flat = _sc_embed(table, idx_hm, n_hist=hist, n_batch=batch)
    out5 = out_flat.reshape(hist, D // 8, batch // UNIT, 8, UNIT)
    return out5.transpose(2, 4, 0, 1, 3).reshape(batch, hist, D)


# speedup vs baseline: 2.1280x; 1.0287x over previous
import functools

import jax
import jax.numpy as jnp
from jax import lax
from jax.experimental import pallas as pl
from jax.experimental.pallas import tpu as pltpu
from jax.experimental.pallas import tpu_sc as plsc

D = 32
NC = 2
NS = 16
NW = NC * NS
UNIT = 128
NBUF = 8


VOCAB_PAD = 1000064
TCOLS = 7812
CPW = 245
RNB = 5


@jax.jit
def _sc_relayout(tt, tail):
    vocab = tt.shape[2]
    mesh = plsc.VectorSubcoreMesh(core_axis_name="c", subcore_axis_name="s")

    @functools.partial(
        pl.kernel,
        out_type=jax.ShapeDtypeStruct((VOCAB_PAD * D,), jnp.float32),
        mesh=mesh,
        scratch_types=(
            [pltpu.VMEM((D // 8, 8, UNIT), jnp.float32)] * RNB
            + [pltpu.VMEM((UNIT * D,), jnp.float32)] * RNB
            + [pltpu.VMEM((64 * D,), jnp.float32)]
            + [pltpu.SemaphoreType.DMA] * (2 * RNB)
        ),
        compiler_params=pltpu.CompilerParams(
            use_tc_tiling_on_sc=True, needs_layout_passes=False
        ),
    )
    def k(tt_hbm, tail_hbm, out_hbm, *rest):
        bufs = rest[:RNB]
        obufs = rest[RNB:2 * RNB]
        tvmem = rest[2 * RNB]
        sems = rest[2 * RNB + 1:]
        gs = sems[:RNB]
        ws = sems[RNB:]
        wid = lax.axis_index("s") * NC + lax.axis_index("c")
        cbase = wid * CPW
        iota = lax.iota(jnp.int32, 16)

        def gathers(ci, s):
            c = cbase + ci
            return [
                pltpu.make_async_copy(
                    tt_hbm.at[pl.ds(e0, 1), :, pl.ds(c * UNIT, UNIT)],
                    bufs[s].at[pl.ds(e0, 1)],
                    gs[s],
                )
                for e0 in range(D // 8)
            ]

        def wcopy(ci, s):
            c = cbase + ci
            return pltpu.make_async_copy(
                obufs[s],
                out_hbm.at[pl.ds(pl.multiple_of(c * UNIT * D, 1024), UNIT * D)],
                ws[s],
            )

        def block(g, carry):
            for s in range(RNB):
                ci = g * RNB + s

                @pl.when((g > 0) & (cbase + ci - RNB < TCOLS))
                def _():
                    wcopy(ci - RNB, s).wait()

                @pl.when(cbase + ci < TCOLS)
                def _():
                    for cp in gathers(ci, s):
                        cp.start()

            for s in range(RNB):
                ci = g * RNB + s

                @pl.when(cbase + ci < TCOLS)
                def _():
                    for cp in gathers(ci, s):
                        cp.wait()
                    rows = bufs[s]
                    obufss = obufs[s]

                    def tr_body(eb, carry2):
                        for ee in range(8):
                            rowv = (iota + eb * 8 + ee) & (D - 1)
                            r_hi = rowv >> 3
                            r_lo = rowv & 7
                            for j in range(UNIT // 16):
                                colv = iota + j * 16
                                v = plsc.load_gather(
                                    rows, [r_hi, r_lo, colv]
                                )
                                plsc.store_scatter(
                                    obufss, [colv * D + rowv], v
                                )
                        return carry2

                    lax.fori_loop(0, D // 8, tr_body, 0)
                    wcopy(ci, s).start()
            return carry

        lax.fori_loop(0, CPW // RNB, block, 0)
        for s in range(RNB):
            ci = (CPW // RNB - 1) * RNB + s

            @pl.when(cbase + ci < TCOLS)
            def _():
                wcopy(ci, s).wait()

        @pl.when(wid == 0)
        def _():
            pltpu.sync_copy(tail_hbm, tvmem)
            pltpu.sync_copy(
                tvmem, out_hbm.at[pl.ds(TCOLS * UNIT * D, 64 * D)]
            )

    return k(tt, tail)


@functools.partial(jax.jit, static_argnames=("n_hist", "n_batch"))
def _sc_embed(table, idx_hm, *, n_hist, n_batch):
    nb0 = n_batch // UNIT
    n_units = n_hist * nb0
    upw = n_units // NW
    slab = D * n_batch
    out_len = n_hist * slab
    n_blocks = upw // NBUF
    mesh = plsc.VectorSubcoreMesh(core_axis_name="c", subcore_axis_name="s")

    @functools.partial(
        pl.kernel,
        out_type=jax.ShapeDtypeStruct((out_len,), jnp.float32),
        mesh=mesh,
        scratch_types=(
            [pltpu.VMEM((upw * UNIT,), jnp.int32),
             pltpu.VMEM((NBUF, UNIT, D), jnp.float32),
             pltpu.VMEM((NBUF, UNIT * D), jnp.float32)]
            + [pltpu.SemaphoreType.DMA] * (2 * NBUF)
        ),
        compiler_params=pltpu.CompilerParams(
            use_tc_tiling_on_sc=False, needs_layout_passes=False
        ),
    )
    def k(table_hbm, idx_hbm, out_hbm, idx_v, rows_v, tbuf, *sems):
        gs = sems[:NBUF]
        ws = sems[NBUF:]
        wid = lax.axis_index("s") * NC + lax.axis_index("c")
        ubase = wid * upw
        pltpu.sync_copy(idx_hbm.at[pl.ds(ubase * UNIT, upw * UNIT)], idx_v)

        iota = lax.iota(jnp.int32, 16)
        rowidx = [iota + j * 16 for j in range(UNIT // 16)]

        def gather(lu, s):
            return pltpu.make_async_copy(
                table_hbm.at[idx_v.at[pl.ds(lu * UNIT, UNIT)]],
                rows_v.at[s],
                gs[s],
            )

        def wcopy(lu, s, e0):
            u = ubase + lu
            h = u // nb0
            b0 = u % nb0
            off = h * slab + (e0 * nb0 + b0) * 1024
            return pltpu.make_async_copy(
                tbuf.at[s, pl.ds(e0 * 1024, 1024)],
                out_hbm.at[pl.ds(pl.multiple_of(off, 1024), 1024)],
                ws[s],
            )

        def block(g, carry):
            for s in range(NBUF):
                lu = g * NBUF + s

                @pl.when(g > 0)
                def _():
                    for e0 in range(D // 8):
                        wcopy(lu - NBUF, s, e0).wait()

                gather(lu, s).start()
            for s in range(NBUF):
                lu = g * NBUF + s
                gather(lu, s).wait()
                rows = rows_v.at[s]
                tbufs = tbuf.at[s]

                def tr_body(eb, carry):
                    for ee in range(8):
                        cole = (iota + eb * 8 + ee) & (D - 1)
                        scol = cole * UNIT + iota
                        loads = [
                            plsc.load_gather(rows, [rowidx[j], cole])
                            for j in range(UNIT // 16)
                        ]
                        for j in range(UNIT // 16):
                            plsc.store_scatter(
                                tbufs, [scol + j * 16], loads[j]
                            )
                    return carry

                lax.fori_loop(0, D // 8, tr_body, 0)
                for e0 in range(D // 8):
                    wcopy(lu, s, e0).start()
            return carry

        lax.fori_loop(0, n_blocks, block, 0)
        for s in range(NBUF):
            for e0 in range(D // 8):
                wcopy((n_blocks - 1) * NBUF + s, s, e0).wait()

    return k(table, idx_hm)


def kernel(inputs, table):
    batch, hist = inputs.shape
    vocab = table.shape[0]
    idx_hm = inputs.T.reshape(batch * hist).astype(jnp.int32)
    tt = table.T.reshape(D // 8, 8, vocab)
    tail = table[vocab - 64:].reshape(64 * D)
    table_rm = _sc_relayout(tt, tail).reshape(VOCAB_PAD, D)
    out_flat = _sc_embed(table_rm, idx_hm, n_hist=hist, n_batch=batch)
    out5 = out_flat.reshape(hist, D // 8, batch // UNIT, 8, UNIT)
    return out5.transpose(2, 4, 0, 1, 3).reshape(batch, hist, D)


# speedup vs baseline: 2.2863x; 1.0744x over previous
import functools

import jax
import jax.numpy as jnp
from jax import lax
from jax.experimental import pallas as pl
from jax.experimental.pallas import tpu as pltpu
from jax.experimental.pallas import tpu_sc as plsc

D = 32
NC = 2
NS = 16
NW = NC * NS
UNIT = 128
NBUF = 8


VOCAB_PAD = 1000064
TCOLS = 7812
CPW = 245
RNB = 5


@jax.jit
def _sc_relayout(tt, tail):
    vocab = tt.shape[2]
    mesh = plsc.VectorSubcoreMesh(core_axis_name="c", subcore_axis_name="s")

    @functools.partial(
        pl.kernel,
        out_type=jax.ShapeDtypeStruct((VOCAB_PAD * D,), jnp.float32),
        mesh=mesh,
        scratch_types=(
            [pltpu.VMEM((D, UNIT), jnp.float32)] * RNB
            + [pltpu.VMEM((UNIT * D,), jnp.float32)] * RNB
            + [pltpu.VMEM((64 * D,), jnp.float32)]
            + [pltpu.SemaphoreType.DMA] * (2 * RNB)
        ),
        compiler_params=pltpu.CompilerParams(
            use_tc_tiling_on_sc=True, needs_layout_passes=False
        ),
    )
    def k(tt_hbm, tail_hbm, out_hbm, *rest):
        bufs = rest[:RNB]
        obufs = rest[RNB:2 * RNB]
        tvmem = rest[2 * RNB]
        sems = rest[2 * RNB + 1:]
        gs = sems[:RNB]
        ws = sems[RNB:]
        wid = lax.axis_index("s") * NC + lax.axis_index("c")
        cbase = wid * CPW
        iota = lax.iota(jnp.int32, 16)
        colvs = [iota + j * 16 for j in range(UNIT // 16)]
        colbase = [(iota + j * 16) * D for j in range(UNIT // 16)]

        def gathers(ci, s):
            c = cbase + ci
            return [
                pltpu.make_async_copy(
                    tt_hbm.at[e0, :, pl.ds(c * UNIT, UNIT)],
                    bufs[s].at[pl.ds(e0 * 8, 8)],
                    gs[s],
                )
                for e0 in range(D // 8)
            ]

        def wcopy(ci, s):
            c = cbase + ci
            return pltpu.make_async_copy(
                obufs[s],
                out_hbm.at[pl.ds(pl.multiple_of(c * UNIT * D, 1024), UNIT * D)],
                ws[s],
            )

        def block(g, carry):
            for s in range(RNB):
                ci = g * RNB + s

                @pl.when((g > 0) & (cbase + ci - RNB < TCOLS))
                def _():
                    wcopy(ci - RNB, s).wait()

                @pl.when(cbase + ci < TCOLS)
                def _():
                    for cp in gathers(ci, s):
                        cp.start()

            for s in range(RNB):
                ci = g * RNB + s

                @pl.when(cbase + ci < TCOLS)
                def _():
                    for cp in gathers(ci, s):
                        cp.wait()
                    rows = bufs[s]
                    obufss = obufs[s]

                    def tr_body(eb, carry2):
                        for ee in range(8):
                            rowv = (iota + eb * 8 + ee) & (D - 1)
                            for j in range(UNIT // 16):
                                v = plsc.load_gather(
                                    rows, [rowv, colvs[j]]
                                )
                                plsc.store_scatter(
                                    obufss, [colbase[j] + rowv], v
                                )
                        return carry2

                    lax.fori_loop(0, D // 8, tr_body, 0)
                    wcopy(ci, s).start()
            return carry

        lax.fori_loop(0, CPW // RNB, block, 0)
        for s in range(RNB):
            ci = (CPW // RNB - 1) * RNB + s

            @pl.when(cbase + ci < TCOLS)
            def _():
                wcopy(ci, s).wait()

        @pl.when(wid == 0)
        def _():
            pltpu.sync_copy(tail_hbm, tvmem)
            pltpu.sync_copy(
                tvmem, out_hbm.at[pl.ds(TCOLS * UNIT * D, 64 * D)]
            )

    return k(tt, tail)


@functools.partial(jax.jit, static_argnames=("n_hist", "n_batch"))
def _sc_embed(table, idx_hm, *, n_hist, n_batch):
    nb0 = n_batch // UNIT
    n_units = n_hist * nb0
    upw = n_units // NW
    slab = D * n_batch
    out_len = n_hist * slab
    n_blocks = upw // NBUF
    mesh = plsc.VectorSubcoreMesh(core_axis_name="c", subcore_axis_name="s")

    @functools.partial(
        pl.kernel,
        out_type=jax.ShapeDtypeStruct((out_len,), jnp.float32),
        mesh=mesh,
        scratch_types=(
            [pltpu.VMEM((upw * UNIT,), jnp.int32),
             pltpu.VMEM((NBUF, UNIT, D), jnp.float32),
             pltpu.VMEM((NBUF, UNIT * D), jnp.float32)]
            + [pltpu.SemaphoreType.DMA] * (2 * NBUF)
        ),
        compiler_params=pltpu.CompilerParams(
            use_tc_tiling_on_sc=False, needs_layout_passes=False
        ),
    )
    def k(table_hbm, idx_hbm, out_hbm, idx_v, rows_v, tbuf, *sems):
        gs = sems[:NBUF]
        ws = sems[NBUF:]
        wid = lax.axis_index("s") * NC + lax.axis_index("c")
        ubase = wid * upw
        pltpu.sync_copy(idx_hbm.at[pl.ds(ubase * UNIT, upw * UNIT)], idx_v)

        iota = lax.iota(jnp.int32, 16)
        rowidx = [iota + j * 16 for j in range(UNIT // 16)]

        def gather(lu, s):
            return pltpu.make_async_copy(
                table_hbm.at[idx_v.at[pl.ds(lu * UNIT, UNIT)]],
                rows_v.at[s],
                gs[s],
            )

        def wcopy(lu, s, e0):
            u = ubase + lu
            h = u // nb0
            b0 = u % nb0
            off = h * slab + (e0 * nb0 + b0) * 1024
            return pltpu.make_async_copy(
                tbuf.at[s, pl.ds(e0 * 1024, 1024)],
                out_hbm.at[pl.ds(pl.multiple_of(off, 1024), 1024)],
                ws[s],
            )

        def block(g, carry):
            for s in range(NBUF):
                lu = g * NBUF + s

                @pl.when(g > 0)
                def _():
                    for e0 in range(D // 8):
                        wcopy(lu - NBUF, s, e0).wait()

                gather(lu, s).start()
            for s in range(NBUF):
                lu = g * NBUF + s
                gather(lu, s).wait()
                rows = rows_v.at[s]
                tbufs = tbuf.at[s]

                def tr_body(eb, carry):
                    for ee in range(8):
                        cole = (iota + eb * 8 + ee) & (D - 1)
                        scol = cole * UNIT + iota
                        loads = [
                            plsc.load_gather(rows, [rowidx[j], cole])
                            for j in range(UNIT // 16)
                        ]
                        for j in range(UNIT // 16):
                            plsc.store_scatter(
                                tbufs, [scol + j * 16], loads[j]
                            )
                    return carry

                lax.fori_loop(0, D // 8, tr_body, 0)
                for e0 in range(D // 8):
                    wcopy(lu, s, e0).start()
            return carry

        lax.fori_loop(0, n_blocks, block, 0)
        for s in range(NBUF):
            for e0 in range(D // 8):
                wcopy((n_blocks - 1) * NBUF + s, s, e0).wait()

    return k(table, idx_hm)


def kernel(inputs, table):
    batch, hist = inputs.shape
    vocab = table.shape[0]
    idx_hm = inputs.T.reshape(batch * hist).astype(jnp.int32)
    tt = table.T.reshape(D // 8, 8, vocab)
    tail = table[vocab - 64:].reshape(64 * D)
    table_rm = _sc_relayout(tt, tail).reshape(VOCAB_PAD, D)
    out_flat = _sc_embed(table_rm, idx_hm, n_hist=hist, n_batch=batch)
    out5 = out_flat.reshape(hist, D // 8, batch // UNIT, 8, UNIT)
    return out5.transpose(2, 4, 0, 1, 3).reshape(batch, hist, D)
